# baseline (speedup 1.0000x reference)
import functools

import jax
import jax.numpy as jnp
from jax.experimental import pallas as pl
from jax.experimental.pallas import tpu as pltpu

_B = 4096
_HID = 2048
_R = 1024
_K = 512
_NS = 110
_ND = 90


def _mlp_pair_kernel(xs_ref, xd_ref, w1s_ref, w1d_ref, b1s_ref, b1d_ref,
                     w2s_ref, w2d_ref, b2s_ref, b2d_ref, ys_ref, yd_ref):
    h = pl.program_id(1)
    hs = jnp.maximum(
        jnp.dot(xs_ref[...], w1s_ref[...], preferred_element_type=jnp.float32)
        + b1s_ref[...], 0.0).astype(w2s_ref.dtype)
    ps = jnp.dot(hs, w2s_ref[...], preferred_element_type=jnp.float32)
    hd = jnp.maximum(
        jnp.dot(xd_ref[...], w1d_ref[...], preferred_element_type=jnp.float32)
        + b1d_ref[...], 0.0).astype(w2d_ref.dtype)
    pd = jnp.dot(hd, w2d_ref[...], preferred_element_type=jnp.float32)

    @pl.when(h == 0)
    def _init():
        ys_ref[...] = ps + b2s_ref[...]
        yd_ref[...] = pd + b2d_ref[...]

    @pl.when(h != 0)
    def _acc():
        ys_ref[...] += ps
        yd_ref[...] += pd


@functools.partial(jax.jit, static_argnames=())
def kernel(features_standard, features_different, subject_id,
           W1_visual, b1_visual, W2_visual, b2_visual,
           W1_dorsattn, b1_dorsattn, W2_dorsattn, b2_dorsattn,
           W1_sommot, b1_sommot, W2_sommot, b2_sommot,
           W1_multi, b1_multi, W2_multi, b2_multi):
    del subject_id
    B = features_standard.shape[0]
    d_std = features_standard.shape[1]
    d_diff = features_different.shape[1]
    n_v, n_do = W2_visual.shape[1], W2_dorsattn.shape[1]
    n_s, n_m = W2_sommot.shape[1], W2_multi.shape[1]
    ns, nd = n_v + n_do, n_s + n_m
    hid = W1_visual.shape[1]
    pair_hid = 2 * hid

    xs = features_standard
    xd = features_different
    w1s = jnp.concatenate([W1_visual, W1_dorsattn], axis=1)
    w1d = jnp.concatenate([W1_sommot, W1_multi], axis=1)
    b1s = jnp.concatenate([b1_visual, b1_dorsattn])[None, :]
    b1d = jnp.concatenate([b1_sommot, b1_multi])[None, :]
    w2s = jnp.zeros((pair_hid, ns), jnp.float32)
    w2s = w2s.at[:hid, :n_v].set(W2_visual).at[hid:, n_v:].set(W2_dorsattn)
    w2d = jnp.zeros((pair_hid, nd), jnp.float32)
    w2d = w2d.at[:hid, :n_s].set(W2_sommot).at[hid:, n_s:].set(W2_multi)
    b2s = jnp.concatenate([b2_visual, b2_dorsattn])[None, :]
    b2d = jnp.concatenate([b2_sommot, b2_multi])[None, :]

    grid = (B // _R, pair_hid // _K)
    ys, yd = pl.pallas_call(
        _mlp_pair_kernel,
        grid=grid,
        in_specs=[
            pl.BlockSpec((_R, d_std), lambda i, h: (i, 0)),
            pl.BlockSpec((_R, d_diff), lambda i, h: (i, 0)),
            pl.BlockSpec((d_std, _K), lambda i, h: (0, h)),
            pl.BlockSpec((d_diff, _K), lambda i, h: (0, h)),
            pl.BlockSpec((1, _K), lambda i, h: (0, h)),
            pl.BlockSpec((1, _K), lambda i, h: (0, h)),
            pl.BlockSpec((_K, ns), lambda i, h: (h, 0)),
            pl.BlockSpec((_K, nd), lambda i, h: (h, 0)),
            pl.BlockSpec((1, ns), lambda i, h: (0, 0)),
            pl.BlockSpec((1, nd), lambda i, h: (0, 0)),
        ],
        out_specs=[
            pl.BlockSpec((_R, ns), lambda i, h: (i, 0)),
            pl.BlockSpec((_R, nd), lambda i, h: (i, 0)),
        ],
        out_shape=[
            jax.ShapeDtypeStruct((B, ns), jnp.float32),
            jax.ShapeDtypeStruct((B, nd), jnp.float32),
        ],
        compiler_params=pltpu.CompilerParams(
            dimension_semantics=("parallel", "arbitrary"),
        ),
    )(xs, xd, w1s, w1d, b1s, b1d, w2s, w2d, b2s, b2d)
    return jnp.concatenate([ys, yd], axis=1)


# speedup vs baseline: 1.0613x; 1.0613x over previous
import functools

import jax
import jax.numpy as jnp
from jax.experimental import pallas as pl
from jax.experimental.pallas import tpu as pltpu

_B = 4096
_HID = 2048
_R = 1024
_K = 1024
_NS = 110
_ND = 90


def _mlp_pair_kernel(xs_ref, xd_ref, w1s_ref, w1d_ref, b1s_ref, b1d_ref,
                     w2s_ref, w2d_ref, b2s_ref, b2d_ref, ys_ref, yd_ref):
    h = pl.program_id(1)
    hs = jnp.maximum(
        jnp.dot(xs_ref[...], w1s_ref[...], preferred_element_type=jnp.float32)
        + b1s_ref[...], 0.0).astype(w2s_ref.dtype)
    ps = jnp.dot(hs, w2s_ref[...], preferred_element_type=jnp.float32)
    hd = jnp.maximum(
        jnp.dot(xd_ref[...], w1d_ref[...], preferred_element_type=jnp.float32)
        + b1d_ref[...], 0.0).astype(w2d_ref.dtype)
    pd = jnp.dot(hd, w2d_ref[...], preferred_element_type=jnp.float32)

    @pl.when(h == 0)
    def _init():
        ys_ref[...] = ps + b2s_ref[...]
        yd_ref[...] = pd + b2d_ref[...]

    @pl.when(h != 0)
    def _acc():
        ys_ref[...] += ps
        yd_ref[...] += pd


@functools.partial(jax.jit, static_argnames=())
def kernel(features_standard, features_different, subject_id,
           W1_visual, b1_visual, W2_visual, b2_visual,
           W1_dorsattn, b1_dorsattn, W2_dorsattn, b2_dorsattn,
           W1_sommot, b1_sommot, W2_sommot, b2_sommot,
           W1_multi, b1_multi, W2_multi, b2_multi):
    del subject_id
    B = features_standard.shape[0]
    d_std = features_standard.shape[1]
    d_diff = features_different.shape[1]
    n_v, n_do = W2_visual.shape[1], W2_dorsattn.shape[1]
    n_s, n_m = W2_sommot.shape[1], W2_multi.shape[1]
    ns, nd = n_v + n_do, n_s + n_m
    hid = W1_visual.shape[1]
    pair_hid = 2 * hid

    xs = features_standard
    xd = features_different
    w1s = jnp.concatenate([W1_visual, W1_dorsattn], axis=1)
    w1d = jnp.concatenate([W1_sommot, W1_multi], axis=1)
    b1s = jnp.concatenate([b1_visual, b1_dorsattn])[None, :]
    b1d = jnp.concatenate([b1_sommot, b1_multi])[None, :]
    w2s = jnp.zeros((pair_hid, ns), jnp.float32)
    w2s = w2s.at[:hid, :n_v].set(W2_visual).at[hid:, n_v:].set(W2_dorsattn)
    w2d = jnp.zeros((pair_hid, nd), jnp.float32)
    w2d = w2d.at[:hid, :n_s].set(W2_sommot).at[hid:, n_s:].set(W2_multi)
    b2s = jnp.concatenate([b2_visual, b2_dorsattn])[None, :]
    b2d = jnp.concatenate([b2_sommot, b2_multi])[None, :]

    grid = (B // _R, pair_hid // _K)
    ys, yd = pl.pallas_call(
        _mlp_pair_kernel,
        grid=grid,
        in_specs=[
            pl.BlockSpec((_R, d_std), lambda i, h: (i, 0)),
            pl.BlockSpec((_R, d_diff), lambda i, h: (i, 0)),
            pl.BlockSpec((d_std, _K), lambda i, h: (0, h)),
            pl.BlockSpec((d_diff, _K), lambda i, h: (0, h)),
            pl.BlockSpec((1, _K), lambda i, h: (0, h)),
            pl.BlockSpec((1, _K), lambda i, h: (0, h)),
            pl.BlockSpec((_K, ns), lambda i, h: (h, 0)),
            pl.BlockSpec((_K, nd), lambda i, h: (h, 0)),
            pl.BlockSpec((1, ns), lambda i, h: (0, 0)),
            pl.BlockSpec((1, nd), lambda i, h: (0, 0)),
        ],
        out_specs=[
            pl.BlockSpec((_R, ns), lambda i, h: (i, 0)),
            pl.BlockSpec((_R, nd), lambda i, h: (i, 0)),
        ],
        out_shape=[
            jax.ShapeDtypeStruct((B, ns), jnp.float32),
            jax.ShapeDtypeStruct((B, nd), jnp.float32),
        ],
        compiler_params=pltpu.CompilerParams(
            dimension_semantics=("parallel", "arbitrary"),
        ),
    )(xs, xd, w1s, w1d, b1s, b1d, w2s, w2d, b2s, b2d)
    return jnp.concatenate([ys, yd], axis=1)


# speedup vs baseline: 1.3591x; 1.2806x over previous
import jax
import jax.numpy as jnp
from jax.experimental import pallas as pl
from jax.experimental.pallas import tpu as pltpu

_R = 1024
_K = 1024


def _pair_kernel(x_ref, w1a_ref, w1b_ref, b1a_ref, b1b_ref,
                 w2a_ref, w2b_ref, b2a_ref, b2b_ref, ya_ref, yb_ref):
    h = pl.program_id(1)
    x = x_ref[...]
    ha = jnp.maximum(
        jnp.dot(x, w1a_ref[...], preferred_element_type=jnp.float32)
        + b1a_ref[...], 0.0)
    pa = jnp.dot(ha, w2a_ref[...], preferred_element_type=jnp.float32)
    hb = jnp.maximum(
        jnp.dot(x, w1b_ref[...], preferred_element_type=jnp.float32)
        + b1b_ref[...], 0.0)
    pb = jnp.dot(hb, w2b_ref[...], preferred_element_type=jnp.float32)

    @pl.when(h == 0)
    def _init():
        ya_ref[...] = pa + b2a_ref[...]
        yb_ref[...] = pb + b2b_ref[...]

    @pl.when(h != 0)
    def _acc():
        ya_ref[...] += pa
        yb_ref[...] += pb


def _pair_mlp(x, w1a, b1a, w2a, b2a, w1b, b1b, w2b, b2b):
    batch, d_in = x.shape
    hid = w1a.shape[1]
    na, nb = w2a.shape[1], w2b.shape[1]
    grid = (batch // _R, hid // _K)
    return pl.pallas_call(
        _pair_kernel,
        grid=grid,
        in_specs=[
            pl.BlockSpec((_R, d_in), lambda i, h: (i, 0)),
            pl.BlockSpec((d_in, _K), lambda i, h: (0, h)),
            pl.BlockSpec((d_in, _K), lambda i, h: (0, h)),
            pl.BlockSpec((1, _K), lambda i, h: (0, h)),
            pl.BlockSpec((1, _K), lambda i, h: (0, h)),
            pl.BlockSpec((_K, na), lambda i, h: (h, 0)),
            pl.BlockSpec((_K, nb), lambda i, h: (h, 0)),
            pl.BlockSpec((1, na), lambda i, h: (0, 0)),
            pl.BlockSpec((1, nb), lambda i, h: (0, 0)),
        ],
        out_specs=[
            pl.BlockSpec((_R, na), lambda i, h: (i, 0)),
            pl.BlockSpec((_R, nb), lambda i, h: (i, 0)),
        ],
        out_shape=[
            jax.ShapeDtypeStruct((batch, na), jnp.float32),
            jax.ShapeDtypeStruct((batch, nb), jnp.float32),
        ],
        compiler_params=pltpu.CompilerParams(
            dimension_semantics=("parallel", "arbitrary"),
        ),
    )(x, w1a, w1b, b1a[None, :], b1b[None, :],
      w2a, w2b, b2a[None, :], b2b[None, :])


def kernel(features_standard, features_different, subject_id,
           W1_visual, b1_visual, W2_visual, b2_visual,
           W1_dorsattn, b1_dorsattn, W2_dorsattn, b2_dorsattn,
           W1_sommot, b1_sommot, W2_sommot, b2_sommot,
           W1_multi, b1_multi, W2_multi, b2_multi):
    del subject_id
    y_v, y_do = _pair_mlp(features_standard,
                          W1_visual, b1_visual, W2_visual, b2_visual,
                          W1_dorsattn, b1_dorsattn, W2_dorsattn, b2_dorsattn)
    y_s, y_m = _pair_mlp(features_different,
                         W1_sommot, b1_sommot, W2_sommot, b2_sommot,
                         W1_multi, b1_multi, W2_multi, b2_multi)
    return jnp.concatenate([y_v, y_do, y_s, y_m], axis=1)


# speedup vs baseline: 1.3608x; 1.0012x over previous
import jax
import jax.numpy as jnp
from jax.experimental import pallas as pl
from jax.experimental.pallas import tpu as pltpu

_R = 2048
_K = 512


def _pair_kernel(x_ref, w1a_ref, w1b_ref, b1a_ref, b1b_ref,
                 w2a_ref, w2b_ref, b2a_ref, b2b_ref, ya_ref, yb_ref):
    h = pl.program_id(1)
    x = x_ref[...]
    ha = jnp.maximum(
        jnp.dot(x, w1a_ref[...], preferred_element_type=jnp.float32)
        + b1a_ref[...], 0.0)
    pa = jnp.dot(ha, w2a_ref[...], preferred_element_type=jnp.float32)
    hb = jnp.maximum(
        jnp.dot(x, w1b_ref[...], preferred_element_type=jnp.float32)
        + b1b_ref[...], 0.0)
    pb = jnp.dot(hb, w2b_ref[...], preferred_element_type=jnp.float32)

    @pl.when(h == 0)
    def _init():
        ya_ref[...] = pa + b2a_ref[...]
        yb_ref[...] = pb + b2b_ref[...]

    @pl.when(h != 0)
    def _acc():
        ya_ref[...] += pa
        yb_ref[...] += pb


def _pair_mlp(x, w1a, b1a, w2a, b2a, w1b, b1b, w2b, b2b):
    batch, d_in = x.shape
    hid = w1a.shape[1]
    na, nb = w2a.shape[1], w2b.shape[1]
    grid = (batch // _R, hid // _K)
    return pl.pallas_call(
        _pair_kernel,
        grid=grid,
        in_specs=[
            pl.BlockSpec((_R, d_in), lambda i, h: (i, 0)),
            pl.BlockSpec((d_in, _K), lambda i, h: (0, h)),
            pl.BlockSpec((d_in, _K), lambda i, h: (0, h)),
            pl.BlockSpec((1, _K), lambda i, h: (0, h)),
            pl.BlockSpec((1, _K), lambda i, h: (0, h)),
            pl.BlockSpec((_K, na), lambda i, h: (h, 0)),
            pl.BlockSpec((_K, nb), lambda i, h: (h, 0)),
            pl.BlockSpec((1, na), lambda i, h: (0, 0)),
            pl.BlockSpec((1, nb), lambda i, h: (0, 0)),
        ],
        out_specs=[
            pl.BlockSpec((_R, na), lambda i, h: (i, 0)),
            pl.BlockSpec((_R, nb), lambda i, h: (i, 0)),
        ],
        out_shape=[
            jax.ShapeDtypeStruct((batch, na), jnp.float32),
            jax.ShapeDtypeStruct((batch, nb), jnp.float32),
        ],
        compiler_params=pltpu.CompilerParams(
            dimension_semantics=("parallel", "arbitrary"),
        ),
    )(x, w1a, w1b, b1a[None, :], b1b[None, :],
      w2a, w2b, b2a[None, :], b2b[None, :])


def kernel(features_standard, features_different, subject_id,
           W1_visual, b1_visual, W2_visual, b2_visual,
           W1_dorsattn, b1_dorsattn, W2_dorsattn, b2_dorsattn,
           W1_sommot, b1_sommot, W2_sommot, b2_sommot,
           W1_multi, b1_multi, W2_multi, b2_multi):
    del subject_id
    y_v, y_do = _pair_mlp(features_standard,
                          W1_visual, b1_visual, W2_visual, b2_visual,
                          W1_dorsattn, b1_dorsattn, W2_dorsattn, b2_dorsattn)
    y_s, y_m = _pair_mlp(features_different,
                         W1_sommot, b1_sommot, W2_sommot, b2_sommot,
                         W1_multi, b1_multi, W2_multi, b2_multi)
    return jnp.concatenate([y_v, y_do, y_s, y_m], axis=1)
